# Initial kernel scaffold; baseline (speedup 1.0000x reference)
import jax
import jax.numpy as jnp
from jax.experimental import pallas as pl


def kernel(atoms, bonds, connectivity, mol_features, X, params):
    raise NotImplementedError("write your pallas kernel here")


# speedup vs baseline: 1.0727x; 1.0727x over previous
import jax
import jax.numpy as jnp
from jax import lax
from jax.experimental import pallas as pl
from jax.experimental.pallas import tpu as pltpu

F = 128
NUM_MESSAGES = 3
ATOM_CLASSES = 64
BOND_CLASSES = 32
NUM_MOLS = 13
B = 2
N_ATOMS = 512
N_BONDS = 1024


def _tower_body(atoms_ref, bonds_ref, conn0_ref, conn1_ref, mf_ref, x_ref,
                aemb_ref, bemb_ref, wg0_ref, bg0_ref,
                wgu1_ref, bgu1_ref, wgu2_ref, bgu2_ref,
                we1_ref, be1_ref, we2_ref, be2_ref,
                wn1_ref, bn1_ref, wn2_ref, bn2_ref,
                wnp1_ref, bnp1_ref, wnp2_ref, bnp2_ref,
                wp_ref, bp_ref, out_ref, acc_ref):
    m = pl.program_id(0)

    @pl.when(m == 0)
    def _init():
        acc_ref[...] = jnp.zeros_like(acc_ref)

    f32 = jnp.float32

    def dg0(a, b):
        return lax.dot_general(a, b, (((0,), (0,)), ((), ())),
                               preferred_element_type=f32, precision=lax.Precision.HIGHEST)

    for b in range(B):
        atom_row = atoms_ref[0, b:b + 1, :]
        ohT_atom = (atom_row == lax.broadcasted_iota(
            jnp.int32, (ATOM_CLASSES, N_ATOMS), 0)).astype(f32)
        bond_row = bonds_ref[0, b:b + 1, :]
        ohT_bond = (bond_row == lax.broadcasted_iota(
            jnp.int32, (BOND_CLASSES, N_BONDS), 0)).astype(f32)
        c0_row = conn0_ref[0, b:b + 1, :]
        ohT_tgt = (c0_row == lax.broadcasted_iota(
            jnp.int32, (N_ATOMS, N_BONDS), 0)).astype(f32)
        c1_row = conn1_ref[0, b:b + 1, :]
        ohT_src = (c1_row == lax.broadcasted_iota(
            jnp.int32, (N_ATOMS, N_BONDS), 0)).astype(f32)

        atom_state = dg0(ohT_atom, aemb_ref[...])
        bond_state = dg0(ohT_bond, bemb_ref[...])

        mf = mf_ref[0][b:b + 1, :]
        gs = jax.nn.relu(jnp.dot(mf, wg0_ref[...],
                                 preferred_element_type=f32, precision=lax.Precision.HIGHEST) + bg0_ref[...])

        for i in range(NUM_MESSAGES):
            g = jnp.mean(atom_state, axis=0, keepdims=True)
            g = jax.nn.relu(jnp.dot(g, wgu1_ref[i],
                                    preferred_element_type=f32, precision=lax.Precision.HIGHEST) + bgu1_ref[i])
            g = jnp.dot(g, wgu2_ref[i], preferred_element_type=f32, precision=lax.Precision.HIGHEST) + bgu2_ref[i]
            gs = gs + g

            src = dg0(ohT_src, atom_state)
            tgt = dg0(ohT_tgt, atom_state)

            we1 = we1_ref[i]
            h = jax.nn.relu(
                jnp.dot(bond_state, we1[0:F], preferred_element_type=f32, precision=lax.Precision.HIGHEST)
                + jnp.dot(src, we1[F:2 * F], preferred_element_type=f32, precision=lax.Precision.HIGHEST)
                + jnp.dot(tgt, we1[2 * F:3 * F], preferred_element_type=f32, precision=lax.Precision.HIGHEST)
                + jnp.dot(gs, we1[3 * F:4 * F], preferred_element_type=f32, precision=lax.Precision.HIGHEST)
                + be1_ref[i])
            bond_state = bond_state + jnp.dot(
                h, we2_ref[i], preferred_element_type=f32, precision=lax.Precision.HIGHEST) + be2_ref[i]

            wn1 = wn1_ref[i]
            hm = jax.nn.relu(
                jnp.dot(src, wn1[0:F], preferred_element_type=f32, precision=lax.Precision.HIGHEST)
                + jnp.dot(bond_state, wn1[F:2 * F], preferred_element_type=f32, precision=lax.Precision.HIGHEST)
                + jnp.dot(gs, wn1[2 * F:3 * F], preferred_element_type=f32, precision=lax.Precision.HIGHEST)
                + bn1_ref[i])
            messages = jnp.dot(hm, wn2_ref[i],
                               preferred_element_type=f32, precision=lax.Precision.HIGHEST) + bn2_ref[i]

            reduced = jnp.dot(ohT_tgt, messages, preferred_element_type=f32, precision=lax.Precision.HIGHEST)

            na = jnp.dot(
                jax.nn.relu(jnp.dot(reduced, wnp1_ref[i],
                                    preferred_element_type=f32, precision=lax.Precision.HIGHEST) + bnp1_ref[i]),
                wnp2_ref[i], preferred_element_type=f32, precision=lax.Precision.HIGHEST) + bnp2_ref[i]
            atom_state = atom_state + na

        pred = jnp.dot(gs, wp_ref[...], preferred_element_type=f32, precision=lax.Precision.HIGHEST) + bp_ref[...]
        xs = x_ref[b, m]
        acc_ref[b:b + 1, :] = acc_ref[b:b + 1, :] + xs * pred

    @pl.when(m == NUM_MOLS - 1)
    def _fin():
        a = acc_ref[...]
        out_ref[...] = a[:, 0:1] * a[:, 2:3] + a[:, 1:2] * a[:, 3:4]


def kernel(atoms, bonds, connectivity, mol_features, X, params):
    blocks = params['blocks']
    stk = lambda name: jnp.stack([blk[name] for blk in blocks])
    stkb = lambda name: jnp.stack([blk[name].reshape(1, -1) for blk in blocks])

    conn0 = connectivity[..., 0]
    conn1 = connectivity[..., 1]

    inputs = [
        atoms, bonds, conn0, conn1, mol_features, X,
        params['atom_emb'], params['bond_emb'],
        params['Wg0'], params['bg0'].reshape(1, F),
        stk('Wgu1'), stkb('bgu1'), stk('Wgu2'), stkb('bgu2'),
        stk('We1'), stkb('be1'), stk('We2'), stkb('be2'),
        stk('Wn1'), stkb('bn1'), stk('Wn2'), stkb('bn2'),
        stk('Wnp1'), stkb('bnp1'), stk('Wnp2'), stkb('bnp2'),
        params['Wp'], params['bp'].reshape(1, 4),
    ]

    def bspec(shape, blocked_lead=False):
        if blocked_lead:
            blk = (1,) + shape[1:]
            return pl.BlockSpec(blk, lambda m: (m,) + (0,) * (len(shape) - 1))
        return pl.BlockSpec(shape, lambda m: (0,) * len(shape))

    in_specs = [
        bspec(atoms.shape, True), bspec(bonds.shape, True),
        bspec(conn0.shape, True), bspec(conn1.shape, True),
        bspec(mol_features.shape, True),
        pl.BlockSpec(memory_space=pltpu.SMEM),
    ] + [bspec(x.shape) for x in inputs[6:]]

    out = pl.pallas_call(
        _tower_body,
        grid=(NUM_MOLS,),
        in_specs=in_specs,
        out_specs=pl.BlockSpec((B, 1), lambda m: (0, 0)),
        out_shape=jax.ShapeDtypeStruct((B, 1), jnp.float32),
        scratch_shapes=[pltpu.VMEM((B, 4), jnp.float32)],
    )(*inputs)
    return out.reshape(B)


# speedup vs baseline: 5.0586x; 4.7156x over previous
import jax
import jax.numpy as jnp
from jax import lax
from jax.experimental import pallas as pl
from jax.experimental.pallas import tpu as pltpu

F = 128
NUM_MESSAGES = 3
ATOM_CLASSES = 64
BOND_CLASSES = 32
NUM_MOLS = 13
B = 2
N_ATOMS = 512
N_BONDS = 1024


def _tower_body(atoms_ref, bonds_ref, conn0_ref, conn1_ref, mf_ref, x_ref,
                aemb_ref, bemb_ref, wg0_ref, bg0_ref,
                wgu1_ref, bgu1_ref, wgu2_ref, bgu2_ref,
                we1_ref, be1_ref, we2_ref, be2_ref,
                wn1_ref, bn1_ref, wn2_ref, bn2_ref,
                wnp1_ref, bnp1_ref, wnp2_ref, bnp2_ref,
                wp_ref, bp_ref, out_ref, acc_ref):
    m = pl.program_id(0)

    @pl.when(m == 0)
    def _init():
        acc_ref[...] = jnp.zeros_like(acc_ref)

    f32 = jnp.float32
    bf16 = jnp.bfloat16
    DG0 = (((0,), (0,)), ((), ()))
    MM = (((1,), (0,)), ((), ()))

    def ohmm(oh_bf, a, dims):
        a_hi = a.astype(bf16)
        a_lo = (a - a_hi.astype(f32)).astype(bf16)
        d = lambda x: lax.dot_general(oh_bf, x, dims,
                                      preferred_element_type=f32)
        return d(a_hi) + d(a_lo)

    for b in range(B):
        atom_row = atoms_ref[0, b:b + 1, :]
        ohT_atom = (atom_row == lax.broadcasted_iota(
            jnp.int32, (ATOM_CLASSES, N_ATOMS), 0)).astype(bf16)
        bond_row = bonds_ref[0, b:b + 1, :]
        ohT_bond = (bond_row == lax.broadcasted_iota(
            jnp.int32, (BOND_CLASSES, N_BONDS), 0)).astype(bf16)
        c0_row = conn0_ref[0, b:b + 1, :]
        ohT_tgt = (c0_row == lax.broadcasted_iota(
            jnp.int32, (N_ATOMS, N_BONDS), 0)).astype(bf16)
        c1_row = conn1_ref[0, b:b + 1, :]
        ohT_src = (c1_row == lax.broadcasted_iota(
            jnp.int32, (N_ATOMS, N_BONDS), 0)).astype(bf16)

        atom_state = ohmm(ohT_atom, aemb_ref[...], DG0)
        bond_state = ohmm(ohT_bond, bemb_ref[...], DG0)

        mf = mf_ref[0][b:b + 1, :]
        gs = jax.nn.relu(jnp.dot(mf, wg0_ref[...],
                                 preferred_element_type=f32, precision=lax.Precision.DEFAULT) + bg0_ref[...])

        for i in range(NUM_MESSAGES):
            g = jnp.mean(atom_state, axis=0, keepdims=True)
            g = jax.nn.relu(jnp.dot(g, wgu1_ref[i],
                                    preferred_element_type=f32, precision=lax.Precision.DEFAULT) + bgu1_ref[i])
            g = jnp.dot(g, wgu2_ref[i], preferred_element_type=f32, precision=lax.Precision.DEFAULT) + bgu2_ref[i]
            gs = gs + g

            as_hi = atom_state.astype(bf16)
            as_lo = (atom_state - as_hi.astype(f32)).astype(bf16)
            dsrc = lambda x: lax.dot_general(ohT_src, x, DG0,
                                             preferred_element_type=f32)
            dtgt = lambda x: lax.dot_general(ohT_tgt, x, DG0,
                                             preferred_element_type=f32)
            src = dsrc(as_hi) + dsrc(as_lo)
            tgt = dtgt(as_hi) + dtgt(as_lo)

            we1 = we1_ref[i]
            h = jax.nn.relu(
                jnp.dot(bond_state, we1[0:F], preferred_element_type=f32, precision=lax.Precision.DEFAULT)
                + jnp.dot(src, we1[F:2 * F], preferred_element_type=f32, precision=lax.Precision.DEFAULT)
                + jnp.dot(tgt, we1[2 * F:3 * F], preferred_element_type=f32, precision=lax.Precision.DEFAULT)
                + jnp.dot(gs, we1[3 * F:4 * F], preferred_element_type=f32, precision=lax.Precision.DEFAULT)
                + be1_ref[i])
            bond_state = bond_state + jnp.dot(
                h, we2_ref[i], preferred_element_type=f32, precision=lax.Precision.DEFAULT) + be2_ref[i]

            wn1 = wn1_ref[i]
            hm = jax.nn.relu(
                jnp.dot(src, wn1[0:F], preferred_element_type=f32, precision=lax.Precision.DEFAULT)
                + jnp.dot(bond_state, wn1[F:2 * F], preferred_element_type=f32, precision=lax.Precision.DEFAULT)
                + jnp.dot(gs, wn1[2 * F:3 * F], preferred_element_type=f32, precision=lax.Precision.DEFAULT)
                + bn1_ref[i])
            messages = jnp.dot(hm, wn2_ref[i],
                               preferred_element_type=f32, precision=lax.Precision.DEFAULT) + bn2_ref[i]

            reduced = ohmm(ohT_tgt, messages, MM)

            na = jnp.dot(
                jax.nn.relu(jnp.dot(reduced, wnp1_ref[i],
                                    preferred_element_type=f32, precision=lax.Precision.DEFAULT) + bnp1_ref[i]),
                wnp2_ref[i], preferred_element_type=f32, precision=lax.Precision.DEFAULT) + bnp2_ref[i]
            atom_state = atom_state + na

        pred = jnp.dot(gs, wp_ref[...], preferred_element_type=f32, precision=lax.Precision.DEFAULT) + bp_ref[...]
        xs = x_ref[b, m]
        acc_ref[b:b + 1, :] = acc_ref[b:b + 1, :] + xs * pred

    @pl.when(m == NUM_MOLS - 1)
    def _fin():
        a = acc_ref[...]
        out_ref[...] = a[:, 0:1] * a[:, 2:3] + a[:, 1:2] * a[:, 3:4]


def kernel(atoms, bonds, connectivity, mol_features, X, params):
    blocks = params['blocks']
    stk = lambda name: jnp.stack([blk[name] for blk in blocks])
    stkb = lambda name: jnp.stack([blk[name].reshape(1, -1) for blk in blocks])

    conn0 = connectivity[..., 0]
    conn1 = connectivity[..., 1]

    inputs = [
        atoms, bonds, conn0, conn1, mol_features, X,
        params['atom_emb'], params['bond_emb'],
        params['Wg0'], params['bg0'].reshape(1, F),
        stk('Wgu1'), stkb('bgu1'), stk('Wgu2'), stkb('bgu2'),
        stk('We1'), stkb('be1'), stk('We2'), stkb('be2'),
        stk('Wn1'), stkb('bn1'), stk('Wn2'), stkb('bn2'),
        stk('Wnp1'), stkb('bnp1'), stk('Wnp2'), stkb('bnp2'),
        params['Wp'], params['bp'].reshape(1, 4),
    ]

    def bspec(shape, blocked_lead=False):
        if blocked_lead:
            blk = (1,) + shape[1:]
            return pl.BlockSpec(blk, lambda m: (m,) + (0,) * (len(shape) - 1))
        return pl.BlockSpec(shape, lambda m: (0,) * len(shape))

    in_specs = [
        bspec(atoms.shape, True), bspec(bonds.shape, True),
        bspec(conn0.shape, True), bspec(conn1.shape, True),
        bspec(mol_features.shape, True),
        pl.BlockSpec(memory_space=pltpu.SMEM),
    ] + [bspec(x.shape) for x in inputs[6:]]

    out = pl.pallas_call(
        _tower_body,
        grid=(NUM_MOLS,),
        in_specs=in_specs,
        out_specs=pl.BlockSpec((B, 1), lambda m: (0, 0)),
        out_shape=jax.ShapeDtypeStruct((B, 1), jnp.float32),
        scratch_shapes=[pltpu.VMEM((B, 4), jnp.float32)],
    )(*inputs)
    return out.reshape(B)


# speedup vs baseline: 6.1473x; 1.2152x over previous
import jax
import jax.numpy as jnp
from jax import lax
from jax.experimental import pallas as pl
from jax.experimental.pallas import tpu as pltpu

F = 128
NUM_MESSAGES = 3
ATOM_CLASSES = 64
BOND_CLASSES = 32
NUM_MOLS = 13
B = 2
N_ATOMS = 512
N_BONDS = 1024

f32 = jnp.float32
bf16 = jnp.bfloat16
DG0 = (((0,), (0,)), ((), ()))


def _hilo(a):
    hi = a.astype(bf16)
    lo = (a - hi.astype(f32)).astype(bf16)
    return jnp.concatenate([hi, lo], axis=0)


def _dot(a, b):
    return jnp.dot(a, b, preferred_element_type=f32)


def _tower_body(atoms_ref, bonds_ref, conn0_ref, conn1_ref, mf_ref, x_ref,
                aemb_ref, bemb_ref, wg0_ref, bg0_ref,
                wgu1_ref, bgu1_ref, wgu2_ref, bgu2_ref,
                we1_ref, be1_ref, we2_ref, be2_ref,
                wn1_ref, bn1_ref, wn2_ref, bn2_ref,
                wnp1_ref, bnp1_ref, wnp2_ref, bnp2_ref,
                wp_ref, bp_ref, out_ref, acc_ref):
    m = pl.program_id(0)

    @pl.when(m == 0)
    def _init():
        acc_ref[...] = jnp.zeros_like(acc_ref)

    N2, E2 = 2 * N_ATOMS, 2 * N_BONDS

    iota_g = lax.broadcasted_iota(jnp.int32, (N2, E2), 0) & (N_ATOMS - 1)
    iota_s0 = lax.broadcasted_iota(jnp.int32, (N_ATOMS, E2), 0)
    iota_a = lax.broadcasted_iota(jnp.int32, (2 * ATOM_CLASSES, N_ATOMS), 0) \
        & (ATOM_CLASSES - 1)
    iota_b = lax.broadcasted_iota(jnp.int32, (2 * BOND_CLASSES, N_BONDS), 0) \
        & (BOND_CLASSES - 1)

    ohg = []
    ohs = []
    a_states = []
    b_states = []
    for b in range(B):
        c0_row = conn0_ref[0, b:b + 1, :]
        c1_row = conn1_ref[0, b:b + 1, :]
        conn_cat = jnp.concatenate([c1_row, c0_row], axis=1)
        ohg.append((conn_cat == iota_g).astype(bf16))
        c0_cat = jnp.concatenate([c0_row, c0_row], axis=1)
        ohs.append((c0_cat == iota_s0).astype(bf16))

        atom_row = atoms_ref[0, b:b + 1, :]
        oha = (atom_row == iota_a).astype(bf16)
        a_states.append(lax.dot_general(oha, aemb_ref[...], DG0,
                                        preferred_element_type=f32))
        bond_row = bonds_ref[0, b:b + 1, :]
        ohb = (bond_row == iota_b).astype(bf16)
        b_states.append(lax.dot_general(ohb, bemb_ref[...], DG0,
                                        preferred_element_type=f32))

    atom_state = jnp.concatenate(a_states, axis=0)
    bond_state = jnp.concatenate(b_states, axis=0)

    gs = jax.nn.relu(_dot(mf_ref[0], wg0_ref[...]) + bg0_ref[...])

    for i in range(NUM_MESSAGES):
        g = jnp.concatenate(
            [jnp.mean(atom_state[b * N_ATOMS:(b + 1) * N_ATOMS], axis=0,
                      keepdims=True) for b in range(B)], axis=0)
        g = jax.nn.relu(_dot(g, wgu1_ref[i]) + bgu1_ref[i])
        g = _dot(g, wgu2_ref[i]) + bgu2_ref[i]
        gs = gs + g

        a_hi = atom_state.astype(bf16)
        a_lo = (atom_state - a_hi.astype(f32)).astype(bf16)
        srcs, tgts = [], []
        for b in range(B):
            rhs = jnp.concatenate([a_hi[b * N_ATOMS:(b + 1) * N_ATOMS],
                                   a_lo[b * N_ATOMS:(b + 1) * N_ATOMS]],
                                  axis=0)
            st = lax.dot_general(ohg[b], rhs, DG0,
                                 preferred_element_type=f32)
            srcs.append(st[0:N_BONDS])
            tgts.append(st[N_BONDS:])
        src = jnp.concatenate(srcs, axis=0)
        tgt = jnp.concatenate(tgts, axis=0)

        we1 = we1_ref[i]
        gterm_e = _dot(gs, we1[3 * F:4 * F]) + be1_ref[i]
        gterm_e = jnp.broadcast_to(
            gterm_e[:, None, :], (B, N_BONDS, 2 * F)).reshape(E2, 2 * F)
        h = jax.nn.relu(_dot(bond_state, we1[0:F])
                        + _dot(src, we1[F:2 * F])
                        + _dot(tgt, we1[2 * F:3 * F])
                        + gterm_e)
        bond_state = bond_state + _dot(h, we2_ref[i]) + be2_ref[i]

        wn1 = wn1_ref[i]
        gterm_n = _dot(gs, wn1[2 * F:3 * F]) + bn1_ref[i]
        gterm_n = jnp.broadcast_to(
            gterm_n[:, None, :], (B, N_BONDS, 2 * F)).reshape(E2, 2 * F)
        hm = jax.nn.relu(_dot(src, wn1[0:F])
                         + _dot(bond_state, wn1[F:2 * F])
                         + gterm_n)
        messages = _dot(hm, wn2_ref[i]) + bn2_ref[i]

        reds = []
        for b in range(B):
            mb = messages[b * N_BONDS:(b + 1) * N_BONDS]
            reds.append(_dot(ohs[b], _hilo(mb)))
        reduced = jnp.concatenate(reds, axis=0)

        na = _dot(jax.nn.relu(_dot(reduced, wnp1_ref[i]) + bnp1_ref[i]),
                  wnp2_ref[i]) + bnp2_ref[i]
        atom_state = atom_state + na

    pred = _dot(gs, wp_ref[...]) + bp_ref[...]
    xcol = jnp.concatenate(
        [jnp.full((1, 1), x_ref[b, m], f32) for b in range(B)], axis=0)
    acc_ref[...] = acc_ref[...] + xcol * pred

    @pl.when(m == NUM_MOLS - 1)
    def _fin():
        a = acc_ref[...]
        out_ref[...] = a[:, 0:1] * a[:, 2:3] + a[:, 1:2] * a[:, 3:4]


def kernel(atoms, bonds, connectivity, mol_features, X, params):
    blocks = params['blocks']
    stk = lambda name: jnp.stack([blk[name] for blk in blocks])
    stkb = lambda name: jnp.stack([blk[name].reshape(1, -1) for blk in blocks])

    conn0 = connectivity[..., 0]
    conn1 = connectivity[..., 1]

    def hilo_np(w):
        hi = w.astype(bf16)
        lo = (w - hi.astype(f32)).astype(bf16)
        return jnp.concatenate([hi, lo], axis=0)

    inputs = [
        atoms, bonds, conn0, conn1, mol_features, X,
        hilo_np(params['atom_emb']), hilo_np(params['bond_emb']),
        params['Wg0'], params['bg0'].reshape(1, F),
        stk('Wgu1'), stkb('bgu1'), stk('Wgu2'), stkb('bgu2'),
        stk('We1'), stkb('be1'), stk('We2'), stkb('be2'),
        stk('Wn1'), stkb('bn1'), stk('Wn2'), stkb('bn2'),
        stk('Wnp1'), stkb('bnp1'), stk('Wnp2'), stkb('bnp2'),
        params['Wp'], params['bp'].reshape(1, 4),
    ]

    def bspec(shape, blocked_lead=False):
        if blocked_lead:
            blk = (1,) + shape[1:]
            return pl.BlockSpec(blk, lambda m: (m,) + (0,) * (len(shape) - 1))
        return pl.BlockSpec(shape, lambda m: (0,) * len(shape))

    in_specs = [
        bspec(atoms.shape, True), bspec(bonds.shape, True),
        bspec(conn0.shape, True), bspec(conn1.shape, True),
        bspec(mol_features.shape, True),
        pl.BlockSpec(memory_space=pltpu.SMEM),
    ] + [bspec(x.shape) for x in inputs[6:]]

    out = pl.pallas_call(
        _tower_body,
        grid=(NUM_MOLS,),
        in_specs=in_specs,
        out_specs=pl.BlockSpec((B, 1), lambda m: (0, 0)),
        out_shape=jax.ShapeDtypeStruct((B, 1), jnp.float32),
        scratch_shapes=[pltpu.VMEM((B, 4), jnp.float32)],
    )(*inputs)
    return out.reshape(B)


# speedup vs baseline: 7.3913x; 1.2024x over previous
import jax
import jax.numpy as jnp
from jax import lax
from jax.experimental import pallas as pl
from jax.experimental.pallas import tpu as pltpu

F = 128
NUM_MESSAGES = 3
ATOM_CLASSES = 64
BOND_CLASSES = 32
NUM_MOLS = 13
B = 2
N_ATOMS = 512
N_BONDS = 1024

f32 = jnp.float32
bf16 = jnp.bfloat16
DG0 = (((0,), (0,)), ((), ()))


def _hilo_cols(a):
    hi = a.astype(bf16)
    lo = (a - hi.astype(f32)).astype(bf16)
    return jnp.concatenate([hi, lo], axis=1)


def _dot(a, b):
    return jnp.dot(a, b, preferred_element_type=f32)


def _tower_body(atoms_ref, bonds_ref, conn0_ref, conn1_ref, mf_ref, x_ref,
                aemb_ref, bemb_ref, wg0_ref, bg0_ref,
                wgu1_ref, bgu1_ref, wgu2_ref, bgu2_ref,
                we1_ref, be1_ref, we2_ref, be2_ref,
                wn1_ref, bn1_ref, wn2_ref, bn2_ref,
                wnp1_ref, bnp1_ref, wnp2_ref, bnp2_ref,
                wp_ref, bp_ref, out_ref, acc_ref):
    m = pl.program_id(0)

    @pl.when(m == 0)
    def _init():
        acc_ref[...] = jnp.zeros_like(acc_ref)

    E2 = 2 * N_BONDS

    iota_g = lax.broadcasted_iota(jnp.int32, (N_ATOMS, E2), 0)
    iota_a = lax.broadcasted_iota(jnp.int32, (ATOM_CLASSES, N_ATOMS), 0)
    iota_b = lax.broadcasted_iota(jnp.int32, (BOND_CLASSES, N_BONDS), 0)

    def halves(x):
        return x[:, 0:F] + x[:, F:2 * F]

    ohg = []
    a_states = []
    b_states = []
    for b in range(B):
        c0_row = conn0_ref[0, b:b + 1, :]
        c1_row = conn1_ref[0, b:b + 1, :]
        conn_cat = jnp.concatenate([c1_row, c0_row], axis=1)
        ohg.append((conn_cat == iota_g).astype(bf16))

        atom_row = atoms_ref[0, b:b + 1, :]
        oha = (atom_row == iota_a).astype(bf16)
        a_states.append(halves(lax.dot_general(
            oha, aemb_ref[...], DG0, preferred_element_type=f32)))
        bond_row = bonds_ref[0, b:b + 1, :]
        ohb = (bond_row == iota_b).astype(bf16)
        b_states.append(halves(lax.dot_general(
            ohb, bemb_ref[...], DG0, preferred_element_type=f32)))

    atom_state = jnp.concatenate(a_states, axis=0)
    bond_state = jnp.concatenate(b_states, axis=0)

    gs = jax.nn.relu(_dot(mf_ref[0], wg0_ref[...]) + bg0_ref[...])

    for i in range(NUM_MESSAGES):
        g = jnp.concatenate(
            [jnp.mean(atom_state[b * N_ATOMS:(b + 1) * N_ATOMS], axis=0,
                      keepdims=True) for b in range(B)], axis=0)
        g = jax.nn.relu(_dot(g, wgu1_ref[i]) + bgu1_ref[i])
        g = _dot(g, wgu2_ref[i]) + bgu2_ref[i]
        gs = gs + g

        a_hl = _hilo_cols(atom_state)
        srcs, tgts = [], []
        for b in range(B):
            st = halves(lax.dot_general(
                ohg[b], a_hl[b * N_ATOMS:(b + 1) * N_ATOMS], DG0,
                preferred_element_type=f32))
            srcs.append(st[0:N_BONDS])
            tgts.append(st[N_BONDS:])
        src = jnp.concatenate(srcs, axis=0)
        tgt = jnp.concatenate(tgts, axis=0)

        we1 = we1_ref[i]
        gterm_e = _dot(gs, we1[3 * F:4 * F]) + be1_ref[i]
        gterm_e = jnp.broadcast_to(
            gterm_e[:, None, :], (B, N_BONDS, 2 * F)).reshape(E2, 2 * F)
        h = jax.nn.relu(_dot(bond_state, we1[0:F])
                        + _dot(src, we1[F:2 * F])
                        + _dot(tgt, we1[2 * F:3 * F])
                        + gterm_e)
        bond_state = bond_state + _dot(h, we2_ref[i]) + be2_ref[i]

        wn1 = wn1_ref[i]
        gterm_n = _dot(gs, wn1[2 * F:3 * F]) + bn1_ref[i]
        gterm_n = jnp.broadcast_to(
            gterm_n[:, None, :], (B, N_BONDS, 2 * F)).reshape(E2, 2 * F)
        hm = jax.nn.relu(_dot(src, wn1[0:F])
                         + _dot(bond_state, wn1[F:2 * F])
                         + gterm_n)
        messages = _dot(hm, wn2_ref[i]) + bn2_ref[i]

        m_hl = _hilo_cols(messages)
        reds = []
        for b in range(B):
            mb = m_hl[b * N_BONDS:(b + 1) * N_BONDS]
            reds.append(halves(_dot(ohg[b][:, N_BONDS:], mb)))
        reduced = jnp.concatenate(reds, axis=0)

        na = _dot(jax.nn.relu(_dot(reduced, wnp1_ref[i]) + bnp1_ref[i]),
                  wnp2_ref[i]) + bnp2_ref[i]
        atom_state = atom_state + na

    pred = _dot(gs, wp_ref[...]) + bp_ref[...]
    xcol = jnp.concatenate(
        [jnp.full((1, 1), x_ref[b, m], f32) for b in range(B)], axis=0)
    acc_ref[...] = acc_ref[...] + xcol * pred

    @pl.when(m == NUM_MOLS - 1)
    def _fin():
        a = acc_ref[...]
        out_ref[...] = a[:, 0:1] * a[:, 2:3] + a[:, 1:2] * a[:, 3:4]


def kernel(atoms, bonds, connectivity, mol_features, X, params):
    blocks = params['blocks']
    stk = lambda name: jnp.stack([blk[name] for blk in blocks])
    stkb = lambda name: jnp.stack([blk[name].reshape(1, -1) for blk in blocks])

    conn0 = connectivity[..., 0]
    conn1 = connectivity[..., 1]

    def hilo_np(w):
        hi = w.astype(bf16)
        lo = (w - hi.astype(f32)).astype(bf16)
        return jnp.concatenate([hi, lo], axis=1)

    inputs = [
        atoms, bonds, conn0, conn1, mol_features, X,
        hilo_np(params['atom_emb']), hilo_np(params['bond_emb']),
        params['Wg0'], params['bg0'].reshape(1, F),
        stk('Wgu1'), stkb('bgu1'), stk('Wgu2'), stkb('bgu2'),
        stk('We1'), stkb('be1'), stk('We2'), stkb('be2'),
        stk('Wn1'), stkb('bn1'), stk('Wn2'), stkb('bn2'),
        stk('Wnp1'), stkb('bnp1'), stk('Wnp2'), stkb('bnp2'),
        params['Wp'], params['bp'].reshape(1, 4),
    ]

    def bspec(shape, blocked_lead=False):
        if blocked_lead:
            blk = (1,) + shape[1:]
            return pl.BlockSpec(blk, lambda m: (m,) + (0,) * (len(shape) - 1))
        return pl.BlockSpec(shape, lambda m: (0,) * len(shape))

    in_specs = [
        bspec(atoms.shape, True), bspec(bonds.shape, True),
        bspec(conn0.shape, True), bspec(conn1.shape, True),
        bspec(mol_features.shape, True),
        pl.BlockSpec(memory_space=pltpu.SMEM),
    ] + [bspec(x.shape) for x in inputs[6:]]

    out = pl.pallas_call(
        _tower_body,
        grid=(NUM_MOLS,),
        in_specs=in_specs,
        out_specs=pl.BlockSpec((B, 1), lambda m: (0, 0)),
        out_shape=jax.ShapeDtypeStruct((B, 1), jnp.float32),
        scratch_shapes=[pltpu.VMEM((B, 4), jnp.float32)],
    )(*inputs)
    return out.reshape(B)


# speedup vs baseline: 7.8146x; 1.0573x over previous
import jax
import jax.numpy as jnp
from jax import lax
from jax.experimental import pallas as pl
from jax.experimental.pallas import tpu as pltpu

F = 128
NUM_MESSAGES = 3
ATOM_CLASSES = 64
BOND_CLASSES = 32
NUM_MOLS = 13
B = 2
T = 2
NT = NUM_MOLS + 1
G = T * B
N_ATOMS = 512
N_BONDS = 1024

f32 = jnp.float32
bf16 = jnp.bfloat16
DG0 = (((0,), (0,)), ((), ()))

_BLK_NAMES = ('Wgu1', 'bgu1', 'Wgu2', 'bgu2', 'We1', 'be1', 'We2', 'be2',
              'Wn1', 'bn1', 'Wn2', 'bn2', 'Wnp1', 'bnp1', 'Wnp2', 'bnp2')


def _hilo_cols(a):
    hi = a.astype(bf16)
    lo = (a - hi.astype(f32)).astype(bf16)
    return jnp.concatenate([hi, lo], axis=1)


def _dot(a, b):
    return jnp.dot(a, b, preferred_element_type=f32)


def _halves(x):
    return x[:, 0:F] + x[:, F:2 * F]


def _tower_body(*refs):
    (atoms_ref, bonds_ref, conn0_ref, conn1_ref, mf_ref, x_ref,
     aemb_ref, bemb_ref, wg0_ref, bg0_ref) = refs[:10]
    blk = [dict(zip(_BLK_NAMES, refs[10 + 16 * i:10 + 16 * (i + 1)]))
           for i in range(NUM_MESSAGES)]
    wp_ref, bp_ref, out_ref, acc_ref = refs[58], refs[59], refs[60], refs[61]

    m = pl.program_id(0)

    @pl.when(m == 0)
    def _init():
        acc_ref[...] = jnp.zeros_like(acc_ref)

    E2 = 2 * N_BONDS

    iota_g = lax.broadcasted_iota(jnp.int32, (N_ATOMS, E2), 0)
    iota_a = lax.broadcasted_iota(jnp.int32, (ATOM_CLASSES, N_ATOMS), 0)
    iota_b = lax.broadcasted_iota(jnp.int32, (BOND_CLASSES, N_BONDS), 0)

    ohg = []
    a_states = []
    b_states = []
    for t in range(T):
        for b in range(B):
            c0_row = conn0_ref[t, b:b + 1, :]
            c1_row = conn1_ref[t, b:b + 1, :]
            conn_cat = jnp.concatenate([c1_row, c0_row], axis=1)
            ohg.append((conn_cat == iota_g).astype(bf16))

            atom_row = atoms_ref[t, b:b + 1, :]
            oha = (atom_row == iota_a).astype(bf16)
            a_states.append(_halves(lax.dot_general(
                oha, aemb_ref[...], DG0, preferred_element_type=f32)))
            bond_row = bonds_ref[t, b:b + 1, :]
            ohb = (bond_row == iota_b).astype(bf16)
            b_states.append(_halves(lax.dot_general(
                ohb, bemb_ref[...], DG0, preferred_element_type=f32)))

    atom_state = jnp.concatenate(a_states, axis=0)
    bond_state = jnp.concatenate(b_states, axis=0)

    mf = mf_ref[...].reshape(G, 2)
    gs = jax.nn.relu(_dot(mf, wg0_ref[...]) + bg0_ref[...])

    for i in range(NUM_MESSAGES):
        bk = blk[i]
        g = jnp.concatenate(
            [jnp.mean(atom_state[k * N_ATOMS:(k + 1) * N_ATOMS], axis=0,
                      keepdims=True) for k in range(G)], axis=0)
        g = jax.nn.relu(_dot(g, bk['Wgu1'][...]) + bk['bgu1'][...])
        g = _dot(g, bk['Wgu2'][...]) + bk['bgu2'][...]
        gs = gs + g

        a_hl = _hilo_cols(atom_state)
        srcs, tgts = [], []
        for k in range(G):
            st = _halves(lax.dot_general(
                ohg[k], a_hl[k * N_ATOMS:(k + 1) * N_ATOMS], DG0,
                preferred_element_type=f32))
            srcs.append(st[0:N_BONDS])
            tgts.append(st[N_BONDS:])
        src = jnp.concatenate(srcs, axis=0)
        tgt = jnp.concatenate(tgts, axis=0)

        we1 = bk['We1']
        gterm_e = _dot(gs, we1[3 * F:4 * F]) + bk['be1'][...]
        gterm_e = jnp.broadcast_to(
            gterm_e[:, None, :], (G, N_BONDS, 2 * F)).reshape(G * N_BONDS,
                                                              2 * F)
        h = jax.nn.relu(_dot(bond_state, we1[0:F])
                        + _dot(src, we1[F:2 * F])
                        + _dot(tgt, we1[2 * F:3 * F])
                        + gterm_e)
        bond_state = bond_state + _dot(h, bk['We2'][...]) + bk['be2'][...]

        wn1 = bk['Wn1']
        gterm_n = _dot(gs, wn1[2 * F:3 * F]) + bk['bn1'][...]
        gterm_n = jnp.broadcast_to(
            gterm_n[:, None, :], (G, N_BONDS, 2 * F)).reshape(G * N_BONDS,
                                                              2 * F)
        hm = jax.nn.relu(_dot(src, wn1[0:F])
                         + _dot(bond_state, wn1[F:2 * F])
                         + gterm_n)
        messages = _dot(hm, bk['Wn2'][...]) + bk['bn2'][...]

        m_hl = _hilo_cols(messages)
        reds = []
        for k in range(G):
            mb = m_hl[k * N_BONDS:(k + 1) * N_BONDS]
            reds.append(_halves(_dot(ohg[k][:, N_BONDS:], mb)))
        reduced = jnp.concatenate(reds, axis=0)

        na = _dot(jax.nn.relu(_dot(reduced, bk['Wnp1'][...]) + bk['bnp1'][...]),
                  bk['Wnp2'][...]) + bk['bnp2'][...]
        atom_state = atom_state + na

    pred = _dot(gs, wp_ref[...]) + bp_ref[...]
    xcol = jnp.concatenate(
        [jnp.full((1, 1), x_ref[b, T * m + t], f32)
         for t in range(T) for b in range(B)], axis=0)
    acc_ref[...] = acc_ref[...] + xcol * pred

    @pl.when(m == NT // T - 1)
    def _fin():
        a = acc_ref[...]
        ab = a[0:B] + a[B:2 * B]
        out_ref[...] = ab[:, 0:1] * ab[:, 2:3] + ab[:, 1:2] * ab[:, 3:4]


def kernel(atoms, bonds, connectivity, mol_features, X, params):
    blocks = params['blocks']

    def pad(x):
        return jnp.concatenate([x, x[:1]], axis=0)

    conn0 = pad(connectivity[..., 0])
    conn1 = pad(connectivity[..., 1])
    Xp = jnp.concatenate([X, jnp.zeros((B, 1), f32)], axis=1)

    def hilo_w(w):
        hi = w.astype(bf16)
        lo = (w - hi.astype(f32)).astype(bf16)
        return jnp.concatenate([hi, lo], axis=1)

    inputs = [
        pad(atoms), pad(bonds), conn0, conn1, pad(mol_features), Xp,
        hilo_w(params['atom_emb']), hilo_w(params['bond_emb']),
        params['Wg0'], params['bg0'].reshape(1, F),
    ]
    for blk in blocks:
        for name in _BLK_NAMES:
            w = blk[name]
            inputs.append(w.reshape(1, -1) if w.ndim == 1 else w)
    inputs += [params['Wp'], params['bp'].reshape(1, 4)]

    def bspec(shape, blocked_lead=False):
        if blocked_lead:
            blk_shape = (T,) + shape[1:]
            return pl.BlockSpec(blk_shape,
                                lambda m: (m,) + (0,) * (len(shape) - 1))
        return pl.BlockSpec(shape, lambda m: (0,) * len(shape))

    in_specs = [
        bspec(inputs[0].shape, True), bspec(inputs[1].shape, True),
        bspec(conn0.shape, True), bspec(conn1.shape, True),
        bspec(inputs[4].shape, True),
        pl.BlockSpec(memory_space=pltpu.SMEM),
    ] + [bspec(x.shape) for x in inputs[6:]]

    out = pl.pallas_call(
        _tower_body,
        grid=(NT // T,),
        in_specs=in_specs,
        out_specs=pl.BlockSpec((B, 1), lambda m: (0, 0)),
        out_shape=jax.ShapeDtypeStruct((B, 1), jnp.float32),
        scratch_shapes=[pltpu.VMEM((G, 4), jnp.float32)],
    )(*inputs)
    return out.reshape(B)
